# baseline (speedup 1.0000x reference)
import functools
import jax
import jax.numpy as jnp
from jax import lax
from jax.experimental import pallas as pl
from jax.experimental.pallas import tpu as pltpu
from jax.experimental.pallas import tpu_sc as plsc

NR = 8192
NZ = 2048
N_QUERY = 1000000

NC = 2
NS = 16
NW = NC * NS

PER_W = 31744
N_PAD = PER_W * NW
NSUB = 8
CHUNK = NSUB * 128
NCHUNK = PER_W // CHUNK
ROWS_W = PER_W // 128


def _body(tab_hbm, r_hbm, z_hbm, out_hbm,
          r_v, z_v, wr_v, wz_v, i00_v, i01_v, i10_v, i11_v,
          t00_v, t01_v, t10_v, t11_v, o_v, sem_g):
    wid = lax.axis_index("c") * NS + lax.axis_index("s")
    row_base = wid * ROWS_W

    @pl.loop(0, NCHUNK)
    def _chunk(c):
        row_off = row_base + c * NSUB
        pltpu.sync_copy(r_hbm.at[pl.ds(row_off, NSUB)], r_v)
        pltpu.sync_copy(z_hbm.at[pl.ds(row_off, NSUB)], z_v)

        @pl.loop(0, NSUB)
        def _idx(j):
            for k in range(8):
                sl = pl.ds(k * 16, 16)
                rv = r_v[j, sl]
                zv = z_v[j, sl]
                ir0 = jnp.minimum(jnp.maximum(rv.astype(jnp.int32), 0), NR - 2)
                iz0 = jnp.minimum(jnp.maximum(zv.astype(jnp.int32), 0), NZ - 2)
                wr_v[j, sl] = rv - ir0.astype(jnp.float32)
                wz_v[j, sl] = zv - iz0.astype(jnp.float32)
                f00 = ir0 * NZ + iz0
                i00_v[j, sl] = f00
                i01_v[j, sl] = f00 + 1
                i10_v[j, sl] = f00 + NZ
                i11_v[j, sl] = f00 + (NZ + 1)

        @pl.loop(0, NSUB)
        def _fire(j):
            pltpu.async_copy(tab_hbm.at[i00_v.at[j]], t00_v.at[j], sem_g)
            pltpu.async_copy(tab_hbm.at[i01_v.at[j]], t01_v.at[j], sem_g)
            pltpu.async_copy(tab_hbm.at[i10_v.at[j]], t10_v.at[j], sem_g)
            pltpu.async_copy(tab_hbm.at[i11_v.at[j]], t11_v.at[j], sem_g)

        @pl.loop(0, NSUB)
        def _drain(j):
            pltpu.make_async_copy(tab_hbm.at[i00_v.at[j]], t00_v.at[j], sem_g).wait()
            pltpu.make_async_copy(tab_hbm.at[i01_v.at[j]], t01_v.at[j], sem_g).wait()
            pltpu.make_async_copy(tab_hbm.at[i10_v.at[j]], t10_v.at[j], sem_g).wait()
            pltpu.make_async_copy(tab_hbm.at[i11_v.at[j]], t11_v.at[j], sem_g).wait()

        @pl.loop(0, NSUB)
        def _mix(j):
            for k in range(8):
                sl = pl.ds(k * 16, 16)
                wr = wr_v[j, sl]
                wz = wz_v[j, sl]
                t00 = t00_v[j, sl]
                t01 = t01_v[j, sl]
                t10 = t10_v[j, sl]
                t11 = t11_v[j, sl]
                a = t00 + wr * (t10 - t00)
                b = t01 + wr * (t11 - t01)
                o_v[j, sl] = a + wz * (b - a)

        pltpu.sync_copy(o_v, out_hbm.at[pl.ds(row_off, NSUB)])


@jax.jit
def _run(r2, z2, tab):
    mesh = plsc.VectorSubcoreMesh(
        core_axis_name="c", subcore_axis_name="s", num_cores=NC, num_subcores=NS
    )
    f = pl.kernel(
        _body,
        out_type=jax.ShapeDtypeStruct((N_PAD // 128, 128), jnp.float32),
        mesh=mesh,
        scratch_types=[
            pltpu.VMEM((NSUB, 128), jnp.float32),
            pltpu.VMEM((NSUB, 128), jnp.float32),
            pltpu.VMEM((NSUB, 128), jnp.float32),
            pltpu.VMEM((NSUB, 128), jnp.float32),
            pltpu.VMEM((NSUB, 128), jnp.int32),
            pltpu.VMEM((NSUB, 128), jnp.int32),
            pltpu.VMEM((NSUB, 128), jnp.int32),
            pltpu.VMEM((NSUB, 128), jnp.int32),
            pltpu.VMEM((NSUB, 128), jnp.float32),
            pltpu.VMEM((NSUB, 128), jnp.float32),
            pltpu.VMEM((NSUB, 128), jnp.float32),
            pltpu.VMEM((NSUB, 128), jnp.float32),
            pltpu.VMEM((NSUB, 128), jnp.float32),
            pltpu.SemaphoreType.DMA,
        ],
    )
    return f(tab, r2, z2)


def kernel(r, z, timetable):
    pad = N_PAD - N_QUERY
    r2 = jnp.pad(r, (0, pad)).reshape(N_PAD // 128, 128)
    z2 = jnp.pad(z, (0, pad)).reshape(N_PAD // 128, 128)
    tab = timetable.reshape(-1)
    out = _run(r2, z2, tab)
    return out.reshape(-1)[:N_QUERY]
